# baseline (speedup 1.0000x reference)
import functools

import jax
import jax.numpy as jnp
from jax import lax
from jax.experimental import pallas as pl
from jax.experimental.pallas import tpu as pltpu
from jax.experimental.pallas import tpu_sc as plsc

HEIGHT = 1000
WIDTH = 1000
NUM_ACTIONS = 4
BATCH = 16384

_NC = 2
_NS = 16
_NW = _NC * _NS
_BPW = BATCH // _NW
_CHUNK = 32
_NCHUNK = _BPW // _CHUNK


def _body(state_hbm, table_hbm, out_hbm, sv, ib, rows, outv, sem):
    wid = lax.axis_index("s") * _NC + lax.axis_index("c")
    base = wid * _BPW

    pltpu.sync_copy(state_hbm.at[pl.ds(base, _BPW), :], sv)

    iota = lax.iota(jnp.int32, 16)
    col0 = jnp.zeros((16,), jnp.int32)
    col1 = jnp.ones((16,), jnp.int32)
    zero = jnp.zeros((16,), jnp.int32)

    copies = []
    for c in range(_NCHUNK):
        ibc = ib.at[c]
        for k in range(_CHUNK // 16):
            ridx = c * _CHUNK + k * 16 + iota
            ys = plsc.load_gather(sv, [ridx, col0])
            xs = plsc.load_gather(sv, [ridx, col1])
            pos = k * 32 + iota * 2
            plsc.store_scatter(ibc, [pos], (ys * WIDTH + xs) * 2)
            plsc.store_scatter(ibc, [pos + 1], zero)
        copies.append(
            pltpu.async_copy(
                table_hbm.at[ibc],
                rows.at[pl.ds(c * 2 * _CHUNK, 2 * _CHUNK), :],
                sem,
            )
        )
    for cp in copies:
        cp.wait()

    cols = [jnp.full((16,), a, jnp.int32) for a in range(NUM_ACTIONS)]
    for t in range(_BPW // 16):
        ridx = 64 * (t // 2) + 16 * (t % 2) + iota
        oidx = t * 16 + iota
        v = [plsc.load_gather(rows, [ridx, cols[a]]) for a in range(NUM_ACTIONS)]
        m = jnp.maximum(jnp.maximum(v[0], v[1]), jnp.maximum(v[2], v[3]))
        e = [jnp.exp(va - m) for va in v]
        s = (e[0] + e[1]) + (e[2] + e[3])
        r = 1.0 / s
        for a in range(NUM_ACTIONS):
            plsc.store_scatter(outv, [oidx, cols[a]], e[a] * r)

    pltpu.sync_copy(outv, out_hbm.at[pl.ds(base, _BPW), :])


@jax.jit
def _run(state, table):
    mesh = plsc.VectorSubcoreMesh(core_axis_name="c", subcore_axis_name="s")
    f = functools.partial(
        pl.kernel,
        mesh=mesh,
        out_type=jax.ShapeDtypeStruct((BATCH, NUM_ACTIONS), jnp.float32),
        compiler_params=pltpu.CompilerParams(
            needs_layout_passes=False, use_tc_tiling_on_sc=False
        ),
        scratch_types=[
            pltpu.VMEM((_BPW, 2), jnp.int32),
            pltpu.VMEM((_NCHUNK, 2 * _CHUNK), jnp.int32),
            pltpu.VMEM((2 * _BPW, NUM_ACTIONS), jnp.float32),
            pltpu.VMEM((_BPW, NUM_ACTIONS), jnp.float32),
            pltpu.SemaphoreType.DMA,
        ],
    )(_body)
    return f(state, table)


def kernel(state, logits):
    state = state.astype(jnp.int32)
    table = logits.reshape(HEIGHT * WIDTH, NUM_ACTIONS)
    return _run(state, table)


# speedup vs baseline: 1.1703x; 1.1703x over previous
import functools

import jax
import jax.numpy as jnp
from jax import lax
from jax.experimental import pallas as pl
from jax.experimental.pallas import tpu as pltpu
from jax.experimental.pallas import tpu_sc as plsc

HEIGHT = 1000
WIDTH = 1000
NUM_ACTIONS = 4
BATCH = 16384

_ROW = 8
_NROW = HEIGHT * WIDTH * NUM_ACTIONS // _ROW

_NC = 2
_NS = 16
_NW = _NC * _NS
_BPW = BATCH // _NW
_CHUNK = 32
_NCHUNK = _BPW // _CHUNK


def _body(state_hbm, table_hbm, out_hbm, sv, ib, sub, rows, outv, sem):
    wid = lax.axis_index("s") * _NC + lax.axis_index("c")
    base = wid * _BPW

    pltpu.sync_copy(state_hbm.at[pl.ds(base, _BPW), :], sv)

    iota = lax.iota(jnp.int32, 16)
    col0 = jnp.zeros((16,), jnp.int32)
    col1 = jnp.ones((16,), jnp.int32)
    zero = jnp.zeros((16,), jnp.int32)

    copies = []
    for c in range(_NCHUNK):
        ibc = ib.at[c]
        for k in range(_CHUNK // 16):
            ridx = c * _CHUNK + k * 16 + iota
            ys = plsc.load_gather(sv, [ridx, col0])
            xs = plsc.load_gather(sv, [ridx, col1])
            f = ys * WIDTH + xs
            pos = k * 32 + iota * 2
            plsc.store_scatter(
                ibc, [pos], lax.shift_left(lax.shift_right_logical(f, 1), 2)
            )
            plsc.store_scatter(ibc, [pos + 1], zero)
            sub[pl.ds(c * _CHUNK + k * 16, 16)] = lax.shift_left(
                jnp.bitwise_and(f, 1), 2
            )
        copies.append(
            pltpu.async_copy(
                table_hbm.at[ibc],
                rows.at[pl.ds(c * 2 * _CHUNK, 2 * _CHUNK), :],
                sem,
            )
        )
    for cp in copies:
        cp.wait()

    cols = [jnp.full((16,), a, jnp.int32) for a in range(NUM_ACTIONS)]
    for t in range(_BPW // 16):
        ridx = 64 * (t // 2) + 16 * (t % 2) + iota
        oidx = t * 16 + iota
        off = sub[pl.ds(t * 16, 16)]
        v = [
            plsc.load_gather(rows, [ridx, off + a])
            for a in range(NUM_ACTIONS)
        ]
        m = jnp.maximum(jnp.maximum(v[0], v[1]), jnp.maximum(v[2], v[3]))
        e = [jnp.exp(va - m) for va in v]
        s = (e[0] + e[1]) + (e[2] + e[3])
        r = 1.0 / s
        for a in range(NUM_ACTIONS):
            plsc.store_scatter(outv, [oidx, cols[a]], e[a] * r)

    pltpu.sync_copy(outv, out_hbm.at[pl.ds(base, _BPW), :])


@jax.jit
def _run(state, table):
    mesh = plsc.VectorSubcoreMesh(core_axis_name="c", subcore_axis_name="s")
    f = functools.partial(
        pl.kernel,
        mesh=mesh,
        out_type=jax.ShapeDtypeStruct((BATCH, NUM_ACTIONS), jnp.float32),
        compiler_params=pltpu.CompilerParams(
            needs_layout_passes=False, use_tc_tiling_on_sc=False
        ),
        scratch_types=[
            pltpu.VMEM((_BPW, 2), jnp.int32),
            pltpu.VMEM((_NCHUNK, 2 * _CHUNK), jnp.int32),
            pltpu.VMEM((_BPW,), jnp.int32),
            pltpu.VMEM((2 * _BPW, _ROW), jnp.float32),
            pltpu.VMEM((_BPW, NUM_ACTIONS), jnp.float32),
            pltpu.SemaphoreType.DMA,
        ],
    )(_body)
    return f(state, table)


def kernel(state, logits):
    state = state.astype(jnp.int32)
    table = logits.reshape(_NROW, _ROW)
    return _run(state, table)


# speedup vs baseline: 1.2332x; 1.0537x over previous
import functools

import jax
import jax.numpy as jnp
from jax import lax
from jax.experimental import pallas as pl
from jax.experimental.pallas import tpu as pltpu
from jax.experimental.pallas import tpu_sc as plsc

HEIGHT = 1000
WIDTH = 1000
NUM_ACTIONS = 4
BATCH = 16384

_ROW = 8
_NROW = HEIGHT * WIDTH * NUM_ACTIONS // _ROW

_NC = 2
_NS = 16
_NW = _NC * _NS
_BPW = BATCH // _NW
_CHUNK = 64
_NCHUNK = _BPW // _CHUNK


def _body(state_hbm, table_hbm, out_hbm, sv, ib, sub, rows, outv, sem):
    wid = lax.axis_index("s") * _NC + lax.axis_index("c")
    base = wid * _BPW

    pltpu.sync_copy(state_hbm.at[pl.ds(base, _BPW), :], sv)

    iota = lax.iota(jnp.int32, 16)
    col0 = jnp.zeros((16,), jnp.int32)
    col1 = jnp.ones((16,), jnp.int32)

    copies = []
    for c in range(_NCHUNK):
        ibc = ib.at[c]
        for k in range(_CHUNK // 16):
            ridx = c * _CHUNK + k * 16 + iota
            ys = plsc.load_gather(sv, [ridx, col0])
            xs = plsc.load_gather(sv, [ridx, col1])
            f = ys * WIDTH + xs
            ibc[pl.ds(k * 16, 16)] = lax.shift_right_logical(f, 1)
            sub[pl.ds(c * _CHUNK + k * 16, 16)] = lax.shift_left(
                jnp.bitwise_and(f, 1), 2
            )
        copies.append(
            pltpu.async_copy(
                table_hbm.at[ibc],
                rows.at[pl.ds(c * _CHUNK, _CHUNK), :],
                sem,
            )
        )
    for cp in copies:
        cp.wait()

    cols = [jnp.full((16,), a, jnp.int32) for a in range(NUM_ACTIONS)]
    for t in range(_BPW // 16):
        ridx = t * 16 + iota
        off = sub[pl.ds(t * 16, 16)]
        v = [
            plsc.load_gather(rows, [ridx, off + a])
            for a in range(NUM_ACTIONS)
        ]
        m = jnp.maximum(jnp.maximum(v[0], v[1]), jnp.maximum(v[2], v[3]))
        e = [jnp.exp(va - m) for va in v]
        s = (e[0] + e[1]) + (e[2] + e[3])
        r = 1.0 / s
        for a in range(NUM_ACTIONS):
            plsc.store_scatter(outv, [ridx, cols[a]], e[a] * r)

    pltpu.sync_copy(outv, out_hbm.at[pl.ds(base, _BPW), :])


@jax.jit
def _run(state, table):
    mesh = plsc.VectorSubcoreMesh(core_axis_name="c", subcore_axis_name="s")
    f = functools.partial(
        pl.kernel,
        mesh=mesh,
        out_type=jax.ShapeDtypeStruct((BATCH, NUM_ACTIONS), jnp.float32),
        compiler_params=pltpu.CompilerParams(
            needs_layout_passes=False, use_tc_tiling_on_sc=False
        ),
        scratch_types=[
            pltpu.VMEM((_BPW, 2), jnp.int32),
            pltpu.VMEM((_NCHUNK, _CHUNK), jnp.int32),
            pltpu.VMEM((_BPW,), jnp.int32),
            pltpu.VMEM((_BPW, _ROW), jnp.float32),
            pltpu.VMEM((_BPW, NUM_ACTIONS), jnp.float32),
            pltpu.SemaphoreType.DMA,
        ],
    )(_body)
    return f(state, table)


def kernel(state, logits):
    state = state.astype(jnp.int32)
    table = logits.reshape(_NROW, _ROW)
    return _run(state, table)


# speedup vs baseline: 1.2361x; 1.0024x over previous
import functools

import jax
import jax.numpy as jnp
from jax import lax
from jax.experimental import pallas as pl
from jax.experimental.pallas import tpu as pltpu
from jax.experimental.pallas import tpu_sc as plsc

HEIGHT = 1000
WIDTH = 1000
NUM_ACTIONS = 4
BATCH = 16384

_ROW = 8
_NROW = HEIGHT * WIDTH * NUM_ACTIONS // _ROW

_NC = 2
_NS = 16
_NW = _NC * _NS
_BPW = BATCH // _NW
_CHUNK = 64
_NCHUNK = _BPW // _CHUNK


def _body(state_hbm, table_hbm, out_hbm, sv, sub, rows, outv, sem):
    wid = lax.axis_index("s") * _NC + lax.axis_index("c")
    base = wid * _BPW

    pltpu.sync_copy(state_hbm.at[pl.ds(base, _BPW), :], sv)

    iota = lax.iota(jnp.int32, 16)
    col0 = jnp.zeros((16,), jnp.int32)
    col1 = jnp.ones((16,), jnp.int32)

    copies = []
    for t in range(_BPW // 16):
        ridx = t * 16 + iota
        ys = plsc.load_gather(sv, [ridx, col0])
        xs = plsc.load_gather(sv, [ridx, col1])
        f = ys * WIDTH + xs
        idx = lax.shift_right_logical(f, 1)
        sub[pl.ds(t * 16, 16)] = lax.shift_left(jnp.bitwise_and(f, 1), 2)
        copies.append(
            pltpu.async_copy(
                table_hbm.at[idx], rows.at[pl.ds(t * 16, 16), :], sem
            )
        )
    for cp in copies:
        cp.wait()

    cols = [jnp.full((16,), a, jnp.int32) for a in range(NUM_ACTIONS)]
    for t in range(_BPW // 16):
        ridx = t * 16 + iota
        off = sub[pl.ds(t * 16, 16)]
        v = [
            plsc.load_gather(rows, [ridx, off + a])
            for a in range(NUM_ACTIONS)
        ]
        m = jnp.maximum(jnp.maximum(v[0], v[1]), jnp.maximum(v[2], v[3]))
        e = [jnp.exp(va - m) for va in v]
        s = (e[0] + e[1]) + (e[2] + e[3])
        r = 1.0 / s
        for a in range(NUM_ACTIONS):
            plsc.store_scatter(outv, [ridx, cols[a]], e[a] * r)

    pltpu.sync_copy(outv, out_hbm.at[pl.ds(base, _BPW), :])


@jax.jit
def _run(state, table):
    mesh = plsc.VectorSubcoreMesh(core_axis_name="c", subcore_axis_name="s")
    f = functools.partial(
        pl.kernel,
        mesh=mesh,
        out_type=jax.ShapeDtypeStruct((BATCH, NUM_ACTIONS), jnp.float32),
        compiler_params=pltpu.CompilerParams(
            needs_layout_passes=False, use_tc_tiling_on_sc=False
        ),
        scratch_types=[
            pltpu.VMEM((_BPW, 2), jnp.int32),
            pltpu.VMEM((_BPW,), jnp.int32),
            pltpu.VMEM((_BPW, _ROW), jnp.float32),
            pltpu.VMEM((_BPW, NUM_ACTIONS), jnp.float32),
            pltpu.SemaphoreType.DMA,
        ],
    )(_body)
    return f(state, table)


def kernel(state, logits):
    state = state.astype(jnp.int32)
    table = logits.reshape(_NROW, _ROW)
    return _run(state, table)


# speedup vs baseline: 1.2404x; 1.0034x over previous
import functools

import jax
import jax.numpy as jnp
from jax import lax
from jax.experimental import pallas as pl
from jax.experimental.pallas import tpu as pltpu
from jax.experimental.pallas import tpu_sc as plsc

HEIGHT = 1000
WIDTH = 1000
NUM_ACTIONS = 4
BATCH = 16384

_ROW = 8
_NROW = HEIGHT * WIDTH * NUM_ACTIONS // _ROW

_NC = 2
_NS = 16
_NW = _NC * _NS
_BPW = BATCH // _NW


def _body(state_hbm, table_hbm, out_hbm, sv, sub, rows, outv, sem):
    wid = lax.axis_index("s") * _NC + lax.axis_index("c")
    base = wid * _BPW

    pltpu.sync_copy(state_hbm.at[pl.ds(base, _BPW), :], sv)

    iota = lax.iota(jnp.int32, 16)
    col0 = jnp.zeros((16,), jnp.int32)
    col1 = jnp.ones((16,), jnp.int32)

    copies = []
    for t in range(_BPW // 16):
        ridx = t * 16 + iota
        ys = plsc.load_gather(sv, [ridx, col0])
        xs = plsc.load_gather(sv, [ridx, col1])
        f = ys * WIDTH + xs
        idx = lax.shift_right_logical(f, 1)
        sub[pl.ds(t * 16, 16)] = lax.shift_left(jnp.bitwise_and(f, 1), 2)
        copies.append(
            pltpu.async_copy(
                table_hbm.at[idx], rows.at[pl.ds(t * 16, 16), :], sem
            )
        )
    for cp in copies:
        cp.wait()

    cols = [jnp.full((16,), a, jnp.int32) for a in range(NUM_ACTIONS)]
    for t in range(_BPW // 16):
        ridx = t * 16 + iota
        off = sub[pl.ds(t * 16, 16)]
        v = [
            plsc.load_gather(rows, [ridx, off + a])
            for a in range(NUM_ACTIONS)
        ]
        m = jnp.maximum(jnp.maximum(v[0], v[1]), jnp.maximum(v[2], v[3]))
        e = [jnp.exp(va - m) for va in v]
        s = (e[0] + e[1]) + (e[2] + e[3])
        r = 1.0 / s
        for a in range(NUM_ACTIONS):
            plsc.store_scatter(outv, [ridx, cols[a]], e[a] * r)

    pltpu.sync_copy(outv, out_hbm.at[pl.ds(base, _BPW), :])


@jax.jit
def _run(state, table):
    mesh = plsc.VectorSubcoreMesh(core_axis_name="c", subcore_axis_name="s")
    f = functools.partial(
        pl.kernel,
        mesh=mesh,
        out_type=jax.ShapeDtypeStruct((BATCH, NUM_ACTIONS), jnp.float32),
        compiler_params=pltpu.CompilerParams(
            needs_layout_passes=False, use_tc_tiling_on_sc=False
        ),
        scratch_types=[
            pltpu.VMEM((_BPW, 2), jnp.int32),
            pltpu.VMEM((_BPW,), jnp.int32),
            pltpu.VMEM((_BPW, _ROW), jnp.float32),
            pltpu.VMEM((_BPW, NUM_ACTIONS), jnp.float32),
            pltpu.SemaphoreType.DMA,
        ],
    )(_body)
    return f(state, table)


def kernel(state, logits):
    state = state.astype(jnp.int32)
    table = logits.reshape(_NROW, _ROW)
    return _run(state, table)
